# baseline (speedup 1.0000x reference)
import functools

import jax
import jax.numpy as jnp
from jax import lax
from jax.experimental import pallas as pl
from jax.experimental.pallas import tpu as pltpu
from jax.experimental.pallas import tpu_sc as plsc

N = 10000
B = 512
FX = 128
FU = 128
H = 128
FU_OUT = 128

BN = 400
G = N // BN

CH = 80
NCHUNK = N // CH
NW = 32
TMAX = (NCHUNK + NW - 1) // NW
LANES = 16


def _tc1_body(x_ref, u_ref, Wk_ref, bk_ref, Wq_ref, bq_ref,
              k_ref, qfull_ref):
    g = pl.program_id(0)
    k_ref[...] = jnp.dot(x_ref[...], Wk_ref[...],
                         preferred_element_type=jnp.float32) + bk_ref[...]

    @pl.when(g == 0)
    def _():
        qfull_ref[...] = jnp.dot(u_ref[...], Wq_ref[...],
                                 preferred_element_type=jnp.float32) + bq_ref[...]


def _tc1(x, u, Wk, bk, Wq, bq):
    return pl.pallas_call(
        _tc1_body,
        grid=(G,),
        in_specs=[
            pl.BlockSpec((BN, FX), lambda g: (g, 0)),
            pl.BlockSpec((B, FU), lambda g: (0, 0)),
            pl.BlockSpec((FX, H), lambda g: (0, 0)),
            pl.BlockSpec((1, H), lambda g: (0, 0)),
            pl.BlockSpec((FU, H), lambda g: (0, 0)),
            pl.BlockSpec((1, H), lambda g: (0, 0)),
        ],
        out_specs=[
            pl.BlockSpec((BN, H), lambda g: (g, 0)),
            pl.BlockSpec((B, H), lambda g: (0, 0)),
        ],
        out_shape=[
            jax.ShapeDtypeStruct((N, H), jnp.float32),
            jax.ShapeDtypeStruct((B, H), jnp.float32),
        ],
        compiler_params=pltpu.CompilerParams(
            dimension_semantics=("arbitrary",)),
    )(x, u, Wk, bk.reshape(1, H), Wq, bq.reshape(1, H))


def _sc_body(x_hbm, k_hbm, qfull_hbm, batch_hbm, zeros_hbm, out_hbm,
             idx_v, k_v, x_v, q_v, acc_sh, sem):
    cid = lax.axis_index("c")
    sid = lax.axis_index("s")
    wid = sid * 2 + cid

    @pl.when(sid == 0)
    def _():
        pltpu.sync_copy(zeros_hbm, acc_sh)

    plsc.subcore_barrier()

    def do_chunk(ci):
        base = ci * CH
        pltpu.sync_copy(batch_hbm.at[ci], idx_v)
        pltpu.sync_copy(k_hbm.at[pl.ds(base, CH)], k_v)
        pltpu.sync_copy(x_hbm.at[pl.ds(base, CH)], x_v)
        pltpu.async_copy(qfull_hbm.at[idx_v], q_v, sem).wait()

        def row(i, _):
            kacc = k_v[i, pl.ds(0, LANES)] * q_v[i, pl.ds(0, LANES)]
            for j in range(1, H // LANES):
                kacc = kacc + (k_v[i, pl.ds(j * LANES, LANES)]
                               * q_v[i, pl.ds(j * LANES, LANES)])
            lane = lax.iota(jnp.int32, LANES)
            for m in (8, 4, 2, 1):
                kacc = kacc + kacc.at[lane ^ m].get(
                    mode="promise_in_bounds")
            av = 1.0 / (1.0 + jnp.exp(-kacc))
            for j in range(FX // LANES):
                x_v[i, pl.ds(j * LANES, LANES)] = (
                    x_v[i, pl.ds(j * LANES, LANES)] * av)
            return _

        lax.fori_loop(0, CH, row, 0)
        pltpu.sync_copy(x_v, acc_sh.at[idx_v], add=True)

    for t in range(TMAX):
        ci = t * NW + wid

        @pl.when(ci < NCHUNK)
        def _():
            do_chunk(ci)

    plsc.subcore_barrier()

    @pl.when(sid == 0)
    def _():
        pltpu.sync_copy(acc_sh, out_hbm.at[cid])


def _sc_stage(x, k, qfull, batch2, zeros):
    mesh = plsc.VectorSubcoreMesh(core_axis_name="c", subcore_axis_name="s")
    f = functools.partial(
        pl.kernel, _sc_body, mesh=mesh,
        out_type=jax.ShapeDtypeStruct((2, B, FX), jnp.float32),
        scratch_types=[
            pltpu.VMEM((CH,), jnp.int32),
            pltpu.VMEM((CH, H), jnp.float32),
            pltpu.VMEM((CH, FX), jnp.float32),
            pltpu.VMEM((CH, H), jnp.float32),
            pltpu.VMEM_SHARED((B, FX), jnp.float32),
            pltpu.SemaphoreType.DMA,
        ],
    )()
    return f(x, k, qfull, batch2, zeros)


def _tc2_body(agg_ref, u_ref, Wu_ref, bu_ref, out_ref):
    xagg = agg_ref[0] + agg_ref[1]
    out_ref[...] = (
        jnp.dot(xagg, Wu_ref[0:FX, :], preferred_element_type=jnp.float32)
        + jnp.dot(u_ref[...], Wu_ref[FX:FX + FU, :],
                  preferred_element_type=jnp.float32)
        + bu_ref[...])


def _tc2(agg, u, Wu, bu):
    return pl.pallas_call(
        _tc2_body,
        out_shape=jax.ShapeDtypeStruct((B, FU_OUT), jnp.float32),
    )(agg, u, Wu, bu.reshape(1, FU_OUT))


def kernel(x, edge_index, e, u, batch, Wk, bk, Wq, bq, Wu, bu):
    del edge_index, e
    batch2 = batch.astype(jnp.int32).reshape(NCHUNK, CH)
    zeros = jnp.zeros((B, FX), jnp.float32)
    k, qfull = _tc1(x, u, Wk, bk, Wq, bq)
    agg = _sc_stage(x, k, qfull, batch2, zeros)
    return _tc2(agg, u, Wu, bu)


# speedup vs baseline: 1.1097x; 1.1097x over previous
import functools

import jax
import jax.numpy as jnp
from jax import lax
from jax.experimental import pallas as pl
from jax.experimental.pallas import tpu as pltpu
from jax.experimental.pallas import tpu_sc as plsc

N = 10000
B = 512
FX = 128
FU = 128
H = 128
FU_OUT = 128

CH = 80
NCHUNK = N // CH
NW = 32
TMAX = (NCHUNK + NW - 1) // NW
LANES = 16
NGRP = CH // LANES


def _tc1_body(u_ref, Wk_ref, bk_ref, Wq_ref, bq_ref, rp_ref):
    qfull = jnp.dot(u_ref[...], Wq_ref[...],
                    preferred_element_type=jnp.float32) + bq_ref[...]
    r = lax.dot_general(qfull, Wk_ref[...], (((1,), (1,)), ((), ())),
                        preferred_element_type=jnp.float32)
    c = lax.dot_general(qfull, bk_ref[...], (((1,), (1,)), ((), ())),
                        preferred_element_type=jnp.float32)
    rp_ref[...] = jnp.concatenate(
        [r, jnp.broadcast_to(c, (B, FX))], axis=1)


def _tc1(u, Wk, bk, Wq, bq):
    return pl.pallas_call(
        _tc1_body,
        out_shape=jax.ShapeDtypeStruct((B, 2 * FX), jnp.float32),
    )(u, Wk, bk.reshape(1, H), Wq, bq.reshape(1, H))


def _sc_body(x_hbm, rp_hbm, batch_hbm, zeros_hbm, out_hbm,
             idx_v, x_v, rp_v, acc_sh, sem):
    cid = lax.axis_index("c")
    sid = lax.axis_index("s")
    wid = sid * 2 + cid

    @pl.when(sid == 0)
    def _():
        pltpu.sync_copy(zeros_hbm, acc_sh)

    plsc.subcore_barrier()

    lane = lax.iota(jnp.int32, LANES)

    def do_chunk(ci):
        base = ci * CH
        pltpu.sync_copy(batch_hbm.at[ci], idx_v)
        pltpu.sync_copy(x_hbm.at[pl.ds(base, CH)], x_v)
        pltpu.async_copy(rp_hbm.at[idx_v], rp_v, sem).wait()

        def row(i, _):
            xs = [x_v[i, pl.ds(j * LANES, LANES)] for j in range(FX // LANES)]
            acc = xs[0] * rp_v[i, pl.ds(0, LANES)]
            for j in range(1, FX // LANES):
                acc = acc + xs[j] * rp_v[i, pl.ds(j * LANES, LANES)]
            for m in (8, 4, 2, 1):
                acc = acc + acc.at[lane ^ m].get(mode="promise_in_bounds")
            sv = acc + rp_v[i, pl.ds(FX, LANES)]
            av = 1.0 / (1.0 + jnp.exp(-sv))
            for j in range(FX // LANES):
                x_v[i, pl.ds(j * LANES, LANES)] = xs[j] * av
            return _

        lax.fori_loop(0, CH, row, 0)
        pltpu.sync_copy(x_v, acc_sh.at[idx_v], add=True)

    for t in range(TMAX):
        ci = t * NW + wid

        @pl.when(ci < NCHUNK)
        def _():
            do_chunk(ci)

    plsc.subcore_barrier()

    @pl.when(sid == 0)
    def _():
        pltpu.sync_copy(acc_sh, out_hbm.at[cid])


def _sc_stage(x, rp, batch2, zeros):
    mesh = plsc.VectorSubcoreMesh(core_axis_name="c", subcore_axis_name="s")
    f = functools.partial(
        pl.kernel, _sc_body, mesh=mesh,
        out_type=jax.ShapeDtypeStruct((2, B, FX), jnp.float32),
        scratch_types=[
            pltpu.VMEM((CH,), jnp.int32),
            pltpu.VMEM((CH, FX), jnp.float32),
            pltpu.VMEM((CH, 2 * FX), jnp.float32),
            pltpu.VMEM_SHARED((B, FX), jnp.float32),
            pltpu.SemaphoreType.DMA,
        ],
    )()
    return f(x, rp, batch2, zeros)


def _tc2_body(agg_ref, u_ref, Wu_ref, bu_ref, out_ref):
    xagg = agg_ref[0] + agg_ref[1]
    out_ref[...] = (
        jnp.dot(xagg, Wu_ref[0:FX, :], preferred_element_type=jnp.float32)
        + jnp.dot(u_ref[...], Wu_ref[FX:FX + FU, :],
                  preferred_element_type=jnp.float32)
        + bu_ref[...])


def _tc2(agg, u, Wu, bu):
    return pl.pallas_call(
        _tc2_body,
        out_shape=jax.ShapeDtypeStruct((B, FU_OUT), jnp.float32),
    )(agg, u, Wu, bu.reshape(1, FU_OUT))


def kernel(x, edge_index, e, u, batch, Wk, bk, Wq, bq, Wu, bu):
    del edge_index, e
    batch2 = batch.astype(jnp.int32).reshape(NCHUNK, CH)
    zeros = jnp.zeros((B, FX), jnp.float32)
    rp = _tc1(u, Wk, bk, Wq, bq)
    agg = _sc_stage(x, rp, batch2, zeros)
    return _tc2(agg, u, Wu, bu)


# speedup vs baseline: 1.3043x; 1.1754x over previous
import functools

import jax
import jax.numpy as jnp
from jax import lax
from jax.experimental import pallas as pl
from jax.experimental.pallas import tpu as pltpu
from jax.experimental.pallas import tpu_sc as plsc

N = 10000
B = 512
FX = 128
FU = 128
H = 128
FU_OUT = 128

CH = 80
NCHUNK = N // CH
NW = 32
TMAX = (NCHUNK + NW - 1) // NW
LANES = 16
NGRP = CH // LANES


def _tc1_body(u_ref, Wk_ref, bk_ref, Wq_ref, bq_ref, rp_ref):
    qfull = jnp.dot(u_ref[...], Wq_ref[...],
                    preferred_element_type=jnp.float32) + bq_ref[...]
    r = lax.dot_general(qfull, Wk_ref[...], (((1,), (1,)), ((), ())),
                        preferred_element_type=jnp.float32)
    c = lax.dot_general(qfull, bk_ref[...], (((1,), (1,)), ((), ())),
                        preferred_element_type=jnp.float32)
    rp_ref[...] = jnp.concatenate(
        [r, jnp.broadcast_to(c, (B, FX))], axis=1)


def _tc1(u, Wk, bk, Wq, bq):
    return pl.pallas_call(
        _tc1_body,
        out_shape=jax.ShapeDtypeStruct((B, 2 * FX), jnp.float32),
    )(u, Wk, bk.reshape(1, H), Wq, bq.reshape(1, H))


def _sc_body(x_hbm, rp_hbm, batch_hbm, zeros_hbm, out_hbm,
             idx_v, x_v, rp_v, acc_sh, sem):
    cid = lax.axis_index("c")
    sid = lax.axis_index("s")
    wid = sid * 2 + cid

    @pl.when(sid == 0)
    def _():
        pltpu.sync_copy(zeros_hbm, acc_sh)

    plsc.subcore_barrier()

    lane = lax.iota(jnp.int32, LANES)

    def do_chunk(ci):
        base = ci * CH
        pltpu.sync_copy(batch_hbm.at[ci], idx_v)
        pltpu.sync_copy(x_hbm.at[pl.ds(base, CH)], x_v)
        pltpu.async_copy(rp_hbm.at[idx_v], rp_v, sem).wait()

        @plsc.parallel_loop(0, CH, 1, unroll=4)
        def row(i):
            xs = [x_v[i, pl.ds(j * LANES, LANES)] for j in range(FX // LANES)]
            acc = xs[0] * rp_v[i, pl.ds(0, LANES)]
            for j in range(1, FX // LANES):
                acc = acc + xs[j] * rp_v[i, pl.ds(j * LANES, LANES)]
            for m in (8, 4, 2, 1):
                acc = acc + acc.at[lane ^ m].get(mode="promise_in_bounds")
            sv = acc + rp_v[i, pl.ds(FX, LANES)]
            av = 1.0 / (1.0 + jnp.exp(-sv))
            for j in range(FX // LANES):
                x_v[i, pl.ds(j * LANES, LANES)] = xs[j] * av
        pltpu.sync_copy(x_v, acc_sh.at[idx_v], add=True)

    for t in range(TMAX):
        ci = t * NW + wid

        @pl.when(ci < NCHUNK)
        def _():
            do_chunk(ci)

    plsc.subcore_barrier()

    @pl.when(sid == 0)
    def _():
        pltpu.sync_copy(acc_sh, out_hbm.at[cid])


def _sc_stage(x, rp, batch2, zeros):
    mesh = plsc.VectorSubcoreMesh(core_axis_name="c", subcore_axis_name="s")
    f = functools.partial(
        pl.kernel, _sc_body, mesh=mesh,
        out_type=jax.ShapeDtypeStruct((2, B, FX), jnp.float32),
        scratch_types=[
            pltpu.VMEM((CH,), jnp.int32),
            pltpu.VMEM((CH, FX), jnp.float32),
            pltpu.VMEM((CH, 2 * FX), jnp.float32),
            pltpu.VMEM_SHARED((B, FX), jnp.float32),
            pltpu.SemaphoreType.DMA,
        ],
    )()
    return f(x, rp, batch2, zeros)


def _tc2_body(agg_ref, u_ref, Wu_ref, bu_ref, out_ref):
    xagg = agg_ref[0] + agg_ref[1]
    out_ref[...] = (
        jnp.dot(xagg, Wu_ref[0:FX, :], preferred_element_type=jnp.float32)
        + jnp.dot(u_ref[...], Wu_ref[FX:FX + FU, :],
                  preferred_element_type=jnp.float32)
        + bu_ref[...])


def _tc2(agg, u, Wu, bu):
    return pl.pallas_call(
        _tc2_body,
        out_shape=jax.ShapeDtypeStruct((B, FU_OUT), jnp.float32),
    )(agg, u, Wu, bu.reshape(1, FU_OUT))


def kernel(x, edge_index, e, u, batch, Wk, bk, Wq, bq, Wu, bu):
    del edge_index, e
    batch2 = batch.astype(jnp.int32).reshape(NCHUNK, CH)
    zeros = jnp.zeros((B, FX), jnp.float32)
    rp = _tc1(u, Wk, bk, Wq, bq)
    agg = _sc_stage(x, rp, batch2, zeros)
    return _tc2(agg, u, Wu, bu)
